# baseline (speedup 1.0000x reference)
import functools

import jax
import jax.numpy as jnp
from jax import lax
from jax.experimental import pallas as pl
from jax.experimental.pallas import tpu as pltpu
from jax.experimental.pallas import tpu_sc as plsc

BATCH = 16384
EMBED_DIM = 32
NUM_CORES = 2
NUM_SUBCORES = 16
LANES = 16
NUM_WORKERS = NUM_CORES * NUM_SUBCORES
B_PER_W = BATCH // NUM_WORKERS
CHUNK = 256
COMPACT_BLK = 8000


def _compact_kernel(t_ref, o_ref):
    x = t_ref[...]
    o_ref[...] = jnp.concatenate(
        [x, jnp.zeros((COMPACT_BLK, 128 - EMBED_DIM), jnp.float32)], axis=1)


def _compact(table):
    num_rows = table.shape[0]
    return pl.pallas_call(
        _compact_kernel,
        grid=(num_rows // COMPACT_BLK,),
        in_specs=[pl.BlockSpec((COMPACT_BLK, EMBED_DIM), lambda i: (i, 0))],
        out_specs=pl.BlockSpec((COMPACT_BLK, 128), lambda i: (i, 0)),
        out_shape=jax.ShapeDtypeStruct((num_rows, 128), jnp.float32),
    )(table)


def _dot_kernel(uid_hbm, iid_hbm, ut_hbm, it_hbm, out_hbm,
                uidx_v, iidx_v, urows_v, irows_v, out_v, sem_u, sem_i):
    wid = lax.axis_index("s") * NUM_CORES + lax.axis_index("c")
    base = pl.multiple_of(wid * B_PER_W, B_PER_W)

    pltpu.sync_copy(uid_hbm.at[pl.ds(base, B_PER_W)], uidx_v)
    pltpu.sync_copy(iid_hbm.at[pl.ds(base, B_PER_W)], iidx_v)

    lane_iota = lax.iota(jnp.int32, LANES)

    def chunk_body(ck, _):
        cbase = pl.multiple_of(ck * CHUNK, CHUNK)
        cp_u = pltpu.async_copy(
            ut_hbm.at[uidx_v.at[pl.ds(cbase, CHUNK)]], urows_v, sem_u)
        cp_i = pltpu.async_copy(
            it_hbm.at[iidx_v.at[pl.ds(cbase, CHUNK)]], irows_v, sem_i)
        cp_u.wait()
        cp_i.wait()

        def block_body(blk, _):
            row_idx = blk * LANES + lane_iota
            acc = jnp.zeros((LANES,), jnp.float32)
            for d in range(EMBED_DIM):
                col_idx = jnp.full((LANES,), d, jnp.int32)
                u = plsc.load_gather(urows_v, [row_idx, col_idx])
                v = plsc.load_gather(irows_v, [row_idx, col_idx])
                acc = acc + u * v
            start = pl.multiple_of(cbase + blk * LANES, LANES)
            out_v[pl.ds(start, LANES)] = acc
            return _

        lax.fori_loop(0, CHUNK // LANES, block_body, None)
        return _

    lax.fori_loop(0, B_PER_W // CHUNK, chunk_body, None)

    pltpu.sync_copy(out_v, out_hbm.at[pl.ds(base, B_PER_W)])


@jax.jit
def _run(user_ids, item_ids, user_table, item_table):
    user_table = _compact(user_table)
    item_table = _compact(item_table)
    mesh = plsc.VectorSubcoreMesh(core_axis_name="c", subcore_axis_name="s")
    return pl.kernel(
        _dot_kernel,
        mesh=mesh,
        out_type=jax.ShapeDtypeStruct((BATCH,), jnp.float32),
        scratch_types=[
            pltpu.VMEM((B_PER_W,), jnp.int32),
            pltpu.VMEM((B_PER_W,), jnp.int32),
            pltpu.VMEM((CHUNK, 128), jnp.float32),
            pltpu.VMEM((CHUNK, 128), jnp.float32),
            pltpu.VMEM((B_PER_W,), jnp.float32),
            pltpu.SemaphoreType.DMA,
            pltpu.SemaphoreType.DMA,
        ],
        compiler_params=pltpu.CompilerParams(
            needs_layout_passes=False, use_tc_tiling_on_sc=False),
    )(user_ids, item_ids, user_table, item_table)


def kernel(user_ids, item_ids, user_table, item_table):
    return _run(user_ids.astype(jnp.int32), item_ids.astype(jnp.int32),
                user_table, item_table)


# speedup vs baseline: 1.3561x; 1.3561x over previous
import functools

import jax
import jax.numpy as jnp
from jax import lax
from jax.experimental import pallas as pl
from jax.experimental.pallas import tpu as pltpu
from jax.experimental.pallas import tpu_sc as plsc

BATCH = 16384
EMBED_DIM = 32
NUM_CORES = 2
NUM_SUBCORES = 16
LANES = 16
NUM_WORKERS = NUM_CORES * NUM_SUBCORES
B_PER_W = BATCH // NUM_WORKERS
CHUNK = 256
COMPACT_BLK = 8000


def _compact_kernel(t_ref, o_ref):
    x = t_ref[...]
    o_ref[...] = jnp.concatenate(
        [x, jnp.zeros((COMPACT_BLK, 128 - EMBED_DIM), jnp.float32)], axis=1)


def _compact(table):
    num_rows = table.shape[0]
    return pl.pallas_call(
        _compact_kernel,
        grid=(num_rows // COMPACT_BLK,),
        in_specs=[pl.BlockSpec((COMPACT_BLK, EMBED_DIM), lambda i: (i, 0))],
        out_specs=pl.BlockSpec((COMPACT_BLK, 128), lambda i: (i, 0)),
        out_shape=jax.ShapeDtypeStruct((num_rows, 128), jnp.float32),
    )(table)


def _dot_kernel(uid_hbm, iid_hbm, ut_hbm, it_hbm, out_hbm,
                uidx_v, iidx_v, urows_v, irows_v, out_v, sem_u, sem_i):
    wid = lax.axis_index("s") * NUM_CORES + lax.axis_index("c")
    base = pl.multiple_of(wid * B_PER_W, B_PER_W)

    pltpu.sync_copy(uid_hbm.at[pl.ds(base, B_PER_W)], uidx_v)
    pltpu.sync_copy(iid_hbm.at[pl.ds(base, B_PER_W)], iidx_v)

    lane_iota = lax.iota(jnp.int32, LANES)

    def chunk_body(ck, _):
        cbase = pl.multiple_of(ck * CHUNK, CHUNK)
        cp_u = pltpu.async_copy(
            ut_hbm.at[uidx_v.at[pl.ds(cbase, CHUNK)]], urows_v, sem_u)
        cp_i = pltpu.async_copy(
            it_hbm.at[iidx_v.at[pl.ds(cbase, CHUNK)]], irows_v, sem_i)
        cp_u.wait()
        cp_i.wait()

        def block_body(blk, _):
            row_idx = blk * LANES + lane_iota
            acc = jnp.zeros((LANES,), jnp.float32)
            for d in range(EMBED_DIM):
                col_idx = jnp.full((LANES,), d, jnp.int32)
                u = plsc.load_gather(urows_v, [row_idx, col_idx])
                v = plsc.load_gather(irows_v, [row_idx, col_idx])
                acc = acc + u * v
            start = pl.multiple_of(cbase + blk * LANES, LANES)
            out_v[pl.ds(start, LANES)] = acc
            return _

        lax.fori_loop(0, CHUNK // LANES, block_body, None)
        return _

    lax.fori_loop(0, B_PER_W // CHUNK, chunk_body, None)

    pltpu.sync_copy(out_v, out_hbm.at[pl.ds(base, B_PER_W)])


@jax.jit
def _run(user_ids, item_ids, user_table, item_table):
    user_table = jnp.pad(user_table, ((0, 0), (0, 128 - EMBED_DIM)))
    item_table = jnp.pad(item_table, ((0, 0), (0, 128 - EMBED_DIM)))
    mesh = plsc.VectorSubcoreMesh(core_axis_name="c", subcore_axis_name="s")
    return pl.kernel(
        _dot_kernel,
        mesh=mesh,
        out_type=jax.ShapeDtypeStruct((BATCH,), jnp.float32),
        scratch_types=[
            pltpu.VMEM((B_PER_W,), jnp.int32),
            pltpu.VMEM((B_PER_W,), jnp.int32),
            pltpu.VMEM((CHUNK, 128), jnp.float32),
            pltpu.VMEM((CHUNK, 128), jnp.float32),
            pltpu.VMEM((B_PER_W,), jnp.float32),
            pltpu.SemaphoreType.DMA,
            pltpu.SemaphoreType.DMA,
        ],
        compiler_params=pltpu.CompilerParams(
            needs_layout_passes=False, use_tc_tiling_on_sc=False),
    )(user_ids, item_ids, user_table, item_table)


def kernel(user_ids, item_ids, user_table, item_table):
    return _run(user_ids.astype(jnp.int32), item_ids.astype(jnp.int32),
                user_table, item_table)


# speedup vs baseline: 2.0516x; 1.5128x over previous
import functools

import jax
import jax.numpy as jnp
from jax import lax
from jax.experimental import pallas as pl
from jax.experimental.pallas import tpu as pltpu
from jax.experimental.pallas import tpu_sc as plsc

BATCH = 16384
EMBED_DIM = 32
NUM_CORES = 2
NUM_SUBCORES = 16
LANES = 16
NUM_WORKERS = NUM_CORES * NUM_SUBCORES
B_PER_W = BATCH // NUM_WORKERS
CHUNK = 256
GROUPS_PER_CHUNK = CHUNK // LANES
NSEM = 1
ROWS_PER_SEM = CHUNK // NSEM


def _dot_kernel(uid_hbm, iid_hbm, ut_hbm, it_hbm, out_hbm,
                uidx_v, iidx_v, urows_v, irows_v, out_v, *sems):
    usems = sems[:NSEM]
    isems = sems[NSEM:]
    wid = lax.axis_index("s") * NUM_CORES + lax.axis_index("c")
    base = pl.multiple_of(wid * B_PER_W, B_PER_W)

    pltpu.sync_copy(uid_hbm.at[pl.ds(base, B_PER_W)], uidx_v)
    pltpu.sync_copy(iid_hbm.at[pl.ds(base, B_PER_W)], iidx_v)

    lane_iota = lax.iota(jnp.int32, LANES)

    def chunk_body(ck, _):
        cbase = pl.multiple_of(ck * CHUNK, CHUNK)

        def issue_body(g, _):
            gstart = pl.multiple_of(cbase + g * LANES, LANES)
            u16 = uidx_v[pl.ds(gstart, LANES)]
            i16 = iidx_v[pl.ds(gstart, LANES)]
            for l in range(LANES):
                j = g * LANES + l
                pltpu.async_copy(ut_hbm.at[u16[l]], urows_v.at[j],
                                 usems[l % NSEM])
                pltpu.async_copy(it_hbm.at[i16[l]], irows_v.at[j],
                                 isems[l % NSEM])
            return _

        lax.fori_loop(0, GROUPS_PER_CHUNK, issue_body, None)

        for k in range(NSEM):
            pltpu.make_async_copy(ut_hbm.at[pl.ds(0, ROWS_PER_SEM)],
                                  urows_v.at[pl.ds(0, ROWS_PER_SEM)],
                                  usems[k]).wait()
            pltpu.make_async_copy(ut_hbm.at[pl.ds(0, ROWS_PER_SEM)],
                                  irows_v.at[pl.ds(0, ROWS_PER_SEM)],
                                  isems[k]).wait()

        def block_body(blk, _):
            row_idx = blk * LANES + lane_iota
            acc = jnp.zeros((LANES,), jnp.float32)
            for d in range(EMBED_DIM):
                col_idx = jnp.full((LANES,), d, jnp.int32)
                u = plsc.load_gather(urows_v, [row_idx, col_idx])
                v = plsc.load_gather(irows_v, [row_idx, col_idx])
                acc = acc + u * v
            start = pl.multiple_of(cbase + blk * LANES, LANES)
            out_v[pl.ds(start, LANES)] = acc
            return _

        lax.fori_loop(0, GROUPS_PER_CHUNK, block_body, None)
        return _

    lax.fori_loop(0, B_PER_W // CHUNK, chunk_body, None)

    pltpu.sync_copy(out_v, out_hbm.at[pl.ds(base, B_PER_W)])


@jax.jit
def _run(user_ids, item_ids, user_table, item_table):
    mesh = plsc.VectorSubcoreMesh(core_axis_name="c", subcore_axis_name="s")
    return pl.kernel(
        _dot_kernel,
        mesh=mesh,
        out_type=jax.ShapeDtypeStruct((BATCH,), jnp.float32),
        scratch_types=[
            pltpu.VMEM((B_PER_W,), jnp.int32),
            pltpu.VMEM((B_PER_W,), jnp.int32),
            pltpu.VMEM((CHUNK, EMBED_DIM), jnp.float32),
            pltpu.VMEM((CHUNK, EMBED_DIM), jnp.float32),
            pltpu.VMEM((B_PER_W,), jnp.float32),
        ] + [pltpu.SemaphoreType.DMA] * (2 * NSEM),
        compiler_params=pltpu.CompilerParams(needs_layout_passes=False),
    )(user_ids, item_ids, user_table, item_table)


def kernel(user_ids, item_ids, user_table, item_table):
    return _run(user_ids.astype(jnp.int32), item_ids.astype(jnp.int32),
                user_table, item_table)


# speedup vs baseline: 2.0550x; 1.0016x over previous
import functools

import jax
import jax.numpy as jnp
from jax import lax
from jax.experimental import pallas as pl
from jax.experimental.pallas import tpu as pltpu
from jax.experimental.pallas import tpu_sc as plsc

BATCH = 16384
EMBED_DIM = 32
NUM_CORES = 2
NUM_SUBCORES = 16
LANES = 16
NUM_WORKERS = NUM_CORES * NUM_SUBCORES
B_PER_W = BATCH // NUM_WORKERS
CHUNK = 256
GROUPS_PER_CHUNK = CHUNK // LANES
NSEM = 1
ROWS_PER_SEM = CHUNK // NSEM


def _dot_kernel(uid_hbm, iid_hbm, ut_hbm, it_hbm, out_hbm,
                uidx_v, iidx_v, urows_v, irows_v, out_v, *sems):
    usems = sems[:NSEM]
    isems = sems[NSEM:]
    wid = lax.axis_index("s") * NUM_CORES + lax.axis_index("c")
    base = pl.multiple_of(wid * B_PER_W, B_PER_W)

    pltpu.sync_copy(uid_hbm.at[pl.ds(base, B_PER_W)], uidx_v)
    pltpu.sync_copy(iid_hbm.at[pl.ds(base, B_PER_W)], iidx_v)

    lane_iota = lax.iota(jnp.int32, LANES)

    def chunk_body(ck, _):
        cbase = pl.multiple_of(ck * CHUNK, CHUNK)

        @plsc.parallel_loop(0, GROUPS_PER_CHUNK, unroll=4)
        def _issue(g):
            gstart = pl.multiple_of(cbase + g * LANES, LANES)
            u16 = uidx_v[pl.ds(gstart, LANES)]
            i16 = iidx_v[pl.ds(gstart, LANES)]
            for l in range(LANES):
                j = g * LANES + l
                pltpu.async_copy(ut_hbm.at[u16[l]], urows_v.at[j],
                                 usems[l % NSEM])
                pltpu.async_copy(it_hbm.at[i16[l]], irows_v.at[j],
                                 isems[l % NSEM])

        for k in range(NSEM):
            pltpu.make_async_copy(ut_hbm.at[pl.ds(0, ROWS_PER_SEM)],
                                  urows_v.at[pl.ds(0, ROWS_PER_SEM)],
                                  usems[k]).wait()
            pltpu.make_async_copy(ut_hbm.at[pl.ds(0, ROWS_PER_SEM)],
                                  irows_v.at[pl.ds(0, ROWS_PER_SEM)],
                                  isems[k]).wait()

        def block_body(blk, _):
            row_idx = blk * LANES + lane_iota
            acc = jnp.zeros((LANES,), jnp.float32)
            for d in range(EMBED_DIM):
                col_idx = jnp.full((LANES,), d, jnp.int32)
                u = plsc.load_gather(urows_v, [row_idx, col_idx])
                v = plsc.load_gather(irows_v, [row_idx, col_idx])
                acc = acc + u * v
            start = pl.multiple_of(cbase + blk * LANES, LANES)
            out_v[pl.ds(start, LANES)] = acc
            return _

        lax.fori_loop(0, GROUPS_PER_CHUNK, block_body, None)
        return _

    lax.fori_loop(0, B_PER_W // CHUNK, chunk_body, None)

    pltpu.sync_copy(out_v, out_hbm.at[pl.ds(base, B_PER_W)])


@jax.jit
def _run(user_ids, item_ids, user_table, item_table):
    mesh = plsc.VectorSubcoreMesh(core_axis_name="c", subcore_axis_name="s")
    return pl.kernel(
        _dot_kernel,
        mesh=mesh,
        out_type=jax.ShapeDtypeStruct((BATCH,), jnp.float32),
        scratch_types=[
            pltpu.VMEM((B_PER_W,), jnp.int32),
            pltpu.VMEM((B_PER_W,), jnp.int32),
            pltpu.VMEM((CHUNK, EMBED_DIM), jnp.float32),
            pltpu.VMEM((CHUNK, EMBED_DIM), jnp.float32),
            pltpu.VMEM((B_PER_W,), jnp.float32),
        ] + [pltpu.SemaphoreType.DMA] * (2 * NSEM),
        compiler_params=pltpu.CompilerParams(needs_layout_passes=False),
    )(user_ids, item_ids, user_table, item_table)


def kernel(user_ids, item_ids, user_table, item_table):
    return _run(user_ids.astype(jnp.int32), item_ids.astype(jnp.int32),
                user_table, item_table)
